# baseline (speedup 1.0000x reference)
import functools

import jax
import jax.numpy as jnp
from jax.experimental import pallas as pl
from jax.experimental.pallas import tpu as pltpu
from jax.experimental.pallas import tpu_sc as plsc

_B, _N, _H, _W = 4, 64, 128, 128
_HW = _H * _W
_C_LAT, _C_CTX, _C_PAY, _HID, _R = 256, 64, 32, 64, 16
_P = 2048
_ROWS = _P // _W
_NBLK = _HW // _P
_NEG = -1e30
_LROW = 128
_NITEM = _B * _NBLK


def _route_body(rows_hbm, valid_hbm, lists_hbm, rows_v, valid_v, list_v):
    wid = jax.lax.axis_index("s") * 2 + jax.lax.axis_index("c")
    for t in range(_NITEM // 32):
        item = wid * (_NITEM // 32) + t
        b = item // _NBLK
        j = item % _NBLK
        pltpu.sync_copy(rows_hbm.at[b], rows_v)
        pltpu.sync_copy(valid_hbm.at[b], valid_v)
        row_lo = _ROWS * j - _R
        row_hi = _ROWS * j + _ROWS - 1 + _R
        zeros = jnp.zeros((16,), jnp.int32)
        for z in range(_LROW // 16):
            list_v[pl.ds(16 * z, 16)] = zeros
        lo_v = jnp.full((16,), row_lo, jnp.int32)
        hi_v = jnp.full((16,), row_hi, jnp.int32)
        for cc in range(_N // 16):
            rv = rows_v[pl.ds(16 * cc, 16)]
            vv = valid_v[pl.ds(16 * cc, 16)]
            act = (rv >= lo_v) & (rv <= hi_v) & (vv > zeros)
            list_v[pl.ds(16 * cc, 16)] = jnp.where(act, 1, 0).astype(jnp.int32)
        pltpu.sync_copy(list_v, lists_hbm.at[b, j])


_route_call = functools.partial(
    pl.kernel,
    mesh=plsc.VectorSubcoreMesh(core_axis_name="c", subcore_axis_name="s"),
    out_type=jax.ShapeDtypeStruct((_B, _NBLK, _LROW), jnp.int32),
    scratch_types=[
        pltpu.VMEM((_N,), jnp.int32),
        pltpu.VMEM((_N,), jnp.int32),
        pltpu.VMEM((_LROW,), jnp.int32),
    ],
)(_route_body)


def _dafusion_block(rows_s, cols_s, list_s, b2_s,
                    Ff_r, qe_r, selev_r, ctx_r, pay_r,
                    cw1_r, cb1_r, cw2_r, cb2_r,
                    pw1_r, pb1_r, pw2_r, pb2_r,
                    qwT_r, qb_r, W1qT_r, W1c_r, G4_r, w2_r, b1_r,
                    out_da, out_cov,
                    S_scr, AP_scr, AC_scr, HP_scr, HP2_scr, list_scr):
    j = pl.program_id(1)

    F_blk = Ff_r[0]
    HqT = jnp.dot(qwT_r[...], F_blk, preferred_element_type=jnp.float32)
    HqT = jax.nn.relu(HqT + qb_r[...])
    g0c = G4_r[0:1, :].reshape(_HID, 1)
    g1c = G4_r[1:2, :].reshape(_HID, 1)
    g2c = G4_r[2:3, :].reshape(_HID, 1)
    g3c = G4_r[3:4, :].reshape(_HID, 1)
    qe_row = qe_r[0]
    AP_scr[...] = (jnp.dot(W1qT_r[...], HqT, preferred_element_type=jnp.float32)
                   + (g3c * 1e-3) * qe_row + b1_r[...])

    ctx = ctx_r[0]
    h_ctx = jax.nn.relu(jnp.dot(ctx, cw1_r[...], preferred_element_type=jnp.float32) + cb1_r[...])
    h_ctx = jax.nn.relu(jnp.dot(h_ctx, cw2_r[...], preferred_element_type=jnp.float32) + cb2_r[...])
    pay = pay_r[0]
    h_pay = jax.nn.relu(jnp.dot(pay, pw1_r[...], preferred_element_type=jnp.float32) + pb1_r[...])
    h_pay = jax.nn.relu(jnp.dot(h_pay, pw2_r[...], preferred_element_type=jnp.float32) + pb2_r[...])
    HP_scr[...] = h_pay
    selev = selev_r[0]
    AC_scr[...] = (jnp.dot(h_ctx, W1c_r[...], preferred_element_type=jnp.float32)
                   - selev * (g3c.reshape(1, _HID) * 1e-3))

    cnt = jnp.int32(0)
    for n in range(_N):
        list_scr[cnt] = n
        cnt = cnt + list_s[0, 0, n]
    count = cnt

    pidx = jax.lax.broadcasted_iota(jnp.int32, (1, _P), 1) + j * _P
    prow = (pidx // _W).astype(jnp.float32)
    pcol = (pidx % _W).astype(jnp.float32)

    w2col = w2_r[...].reshape(_HID, 1)
    b2 = b2_s[0, 0]
    r2 = jnp.float32(_R * _R)

    S_scr[...] = jnp.full((_N, _P), _NEG, jnp.float32)
    HP2_scr[...] = jnp.zeros((_N, _HID), jnp.float32)

    AP = AP_scr[...]

    def _score(i):
        n = list_scr[i]
        rn = rows_s[0, 0, n].astype(jnp.float32)
        cn = cols_s[0, 0, n].astype(jnp.float32)
        drf = prow - rn
        dcf = pcol - cn
        d2 = drf * drf + dcf * dcf
        d = jnp.sqrt(d2)
        ok = d2 <= r2
        dkm = jnp.maximum(d, 1e-6)
        inv = 1.0 / dkm
        sinb = dcf * inv
        cosb = jnp.where(d > 0, -drf * inv, -1.0)
        acol = AC_scr[pl.ds(n, 1), :].reshape(_HID, 1)
        pre = (AP + acol
               + g0c * (dkm * (1.0 / _R)) + g1c * sinb + g2c * cosb)
        s = jnp.sum(jax.nn.relu(pre) * w2col, axis=0, keepdims=True) + b2
        return n, jnp.where(ok, s, _NEG)

    _U = 2

    def _stationU(it, carry):
        base = _U * it
        res = [_score(base + k) for k in range(_U)]
        n0, sm0 = res[0]
        S_scr[pl.ds(base, 1), :] = sm0
        HP2_scr[pl.ds(base, 1), :] = HP_scr[pl.ds(n0, 1), :]
        for k in range(1, _U):
            ik = base + k
            nk, smk = res[k]

            @pl.when(ik < count)
            def _(ik=ik, nk=nk, smk=smk):
                S_scr[pl.ds(ik, 1), :] = smk
                HP2_scr[pl.ds(ik, 1), :] = HP_scr[pl.ds(nk, 1), :]

        return carry

    jax.lax.fori_loop(0, (count + _U - 1) // _U, _stationU, jnp.int32(0))

    S = S_scr[...]
    m_raw = jnp.max(S, axis=0, keepdims=True)
    m = jnp.where(m_raw > 0.5 * _NEG, m_raw, 0.0)
    E = jnp.where(S > 0.5 * _NEG, jnp.exp(S - m), 0.0)
    den = jnp.sum(E, axis=0, keepdims=True)
    NumT = jax.lax.dot_general(HP2_scr[...], E, (((0,), (0,)), ((), ())),
                               preferred_element_type=jnp.float32)
    out_da[0] = NumT / jnp.maximum(den, 1e-30)
    out_cov[0] = (den > 0.5).astype(jnp.float32)


def kernel(F_grid, src_rows, src_cols, src_ctx, src_pay, src_valid,
           raw_elev_patch, src_elev,
           ctx_w1, ctx_b1, ctx_w2, ctx_b2, pay_w1, pay_b1, pay_w2, pay_b2,
           q_w, q_b, att_w1, att_b1, att_w2, att_b2):
    Ff = F_grid.reshape(_B, _C_LAT, _HW)
    qe = raw_elev_patch.reshape(_B, 1, _HW)
    rows2d = src_rows.astype(jnp.int32)
    valid2d = src_valid.astype(jnp.int32)
    lists = _route_call(rows2d, valid2d)
    lists3 = lists.reshape(_B, 1, _NBLK * _LROW)
    rows_i = rows2d.reshape(_B, 1, _N)
    cols_i = src_cols.astype(jnp.int32).reshape(_B, 1, _N)
    selev = src_elev.reshape(_B, _N, 1)
    qwT = q_w.T
    W1qT = att_w1[:_HID].T
    W1c = att_w1[_HID:2 * _HID]
    G4 = att_w1[2 * _HID:]
    w2row = att_w2.reshape(1, _HID)
    b1col = att_b1.reshape(_HID, 1)
    b2s = att_b2.reshape(1, 1)
    qbcol = q_b.reshape(_HID, 1)
    cb1r, cb2r = ctx_b1.reshape(1, _HID), ctx_b2.reshape(1, _HID)
    pb1r, pb2r = pay_b1.reshape(1, _HID), pay_b2.reshape(1, _HID)

    smem = pltpu.MemorySpace.SMEM
    grid = (_B, _NBLK)

    def full(shape):
        return pl.BlockSpec(shape, lambda b, j: (0,) * len(shape))

    in_specs = [
        pl.BlockSpec((1, 1, _N), lambda b, j: (b, 0, 0), memory_space=smem),
        pl.BlockSpec((1, 1, _N), lambda b, j: (b, 0, 0), memory_space=smem),
        pl.BlockSpec((1, 1, _LROW), lambda b, j: (b, 0, j), memory_space=smem),
        pl.BlockSpec((1, 1), lambda b, j: (0, 0), memory_space=smem),
        pl.BlockSpec((1, _C_LAT, _P), lambda b, j: (b, 0, j)),
        pl.BlockSpec((1, 1, _P), lambda b, j: (b, 0, j)),
        pl.BlockSpec((1, _N, 1), lambda b, j: (b, 0, 0)),
        pl.BlockSpec((1, _N, _C_CTX), lambda b, j: (b, 0, 0)),
        pl.BlockSpec((1, _N, _C_PAY), lambda b, j: (b, 0, 0)),
        full((_C_CTX, _HID)), full((1, _HID)),
        full((_HID, _HID)), full((1, _HID)),
        full((_C_PAY, _HID)), full((1, _HID)),
        full((_HID, _HID)), full((1, _HID)),
        full((_HID, _C_LAT)), full((_HID, 1)),
        full((_HID, _HID)), full((_HID, _HID)),
        full((4, _HID)), full((1, _HID)), full((_HID, 1)),
    ]
    out_specs = [
        pl.BlockSpec((1, _HID, _P), lambda b, j: (b, 0, j)),
        pl.BlockSpec((1, 1, _P), lambda b, j: (b, 0, j)),
    ]
    out_shape = [
        jax.ShapeDtypeStruct((_B, _HID, _HW), jnp.float32),
        jax.ShapeDtypeStruct((_B, 1, _HW), jnp.float32),
    ]

    da_flat, cov_flat = pl.pallas_call(
        _dafusion_block,
        grid=grid,
        in_specs=in_specs,
        out_specs=out_specs,
        out_shape=out_shape,
        scratch_shapes=[
            pltpu.VMEM((_N, _P), jnp.float32),
            pltpu.VMEM((_HID, _P), jnp.float32),
            pltpu.VMEM((_N, _HID), jnp.float32),
            pltpu.VMEM((_N, _HID), jnp.float32),
            pltpu.VMEM((_N, _HID), jnp.float32),
            pltpu.SMEM((_N,), jnp.int32),
        ],
        compiler_params=pltpu.CompilerParams(
            dimension_semantics=("parallel", "arbitrary")),
    )(rows_i, cols_i, lists3, b2s, Ff, qe, selev, src_ctx, src_pay,
      ctx_w1, cb1r, ctx_w2, cb2r, pay_w1, pb1r, pay_w2, pb2r,
      qwT, qbcol, W1qT, W1c, G4, w2row, b1col)

    da = da_flat.reshape(_B, _HID, _H, _W)
    cov = cov_flat.reshape(_B, 1, _H, _W)
    return da, cov


# speedup vs baseline: 1.2699x; 1.2699x over previous
import functools

import jax
import jax.numpy as jnp
from jax.experimental import pallas as pl
from jax.experimental.pallas import tpu as pltpu
from jax.experimental.pallas import tpu_sc as plsc

_B, _N, _H, _W = 4, 64, 128, 128
_HW = _H * _W
_C_LAT, _C_CTX, _C_PAY, _HID, _R = 256, 64, 32, 64, 16
_P = 1024
_ROWS = _P // _W
_NBLK = _HW // _P
_NEG = -1e30
_LROW = 128
_NITEM = _B * _NBLK


def _route_body(rows_hbm, valid_hbm, lists_hbm, rows_v, valid_v, list_v):
    wid = jax.lax.axis_index("s") * 2 + jax.lax.axis_index("c")
    for t in range(_NITEM // 32):
        item = wid * (_NITEM // 32) + t
        b = item // _NBLK
        j = item % _NBLK
        pltpu.sync_copy(rows_hbm.at[b], rows_v)
        pltpu.sync_copy(valid_hbm.at[b], valid_v)
        row_lo = _ROWS * j - _R
        row_hi = _ROWS * j + _ROWS - 1 + _R
        zeros = jnp.zeros((16,), jnp.int32)
        for z in range(_LROW // 16):
            list_v[pl.ds(16 * z, 16)] = zeros
        lo_v = jnp.full((16,), row_lo, jnp.int32)
        hi_v = jnp.full((16,), row_hi, jnp.int32)
        for cc in range(_N // 16):
            rv = rows_v[pl.ds(16 * cc, 16)]
            vv = valid_v[pl.ds(16 * cc, 16)]
            act = (rv >= lo_v) & (rv <= hi_v) & (vv > zeros)
            list_v[pl.ds(16 * cc, 16)] = jnp.where(act, 1, 0).astype(jnp.int32)
        pltpu.sync_copy(list_v, lists_hbm.at[b, j])


_route_call = functools.partial(
    pl.kernel,
    mesh=plsc.VectorSubcoreMesh(core_axis_name="c", subcore_axis_name="s"),
    out_type=jax.ShapeDtypeStruct((_B, _NBLK, _LROW), jnp.int32),
    scratch_types=[
        pltpu.VMEM((_N,), jnp.int32),
        pltpu.VMEM((_N,), jnp.int32),
        pltpu.VMEM((_LROW,), jnp.int32),
    ],
)(_route_body)


def _dafusion_block(rows_s, cols_s, list_s, b2_s,
                    Ff_r, qe_r, selev_r, ctx_r, pay_r,
                    cw1_r, cb1_r, cw2_r, cb2_r,
                    pw1_r, pb1_r, pw2_r, pb2_r,
                    qwT_r, qb_r, W1qT_r, W1c_r, G4_r, w2_r, b1_r,
                    out_da, out_cov,
                    S_scr, AP_scr, AC_scr, HP_scr, HP2_scr, list_scr):
    j = pl.program_id(1)

    F_blk = Ff_r[0]
    HqT = jnp.dot(qwT_r[...], F_blk, preferred_element_type=jnp.float32)
    HqT = jax.nn.relu(HqT + qb_r[...])
    g0c = G4_r[0:1, :].reshape(_HID, 1)
    g1c = G4_r[1:2, :].reshape(_HID, 1)
    g2c = G4_r[2:3, :].reshape(_HID, 1)
    g3c = G4_r[3:4, :].reshape(_HID, 1)
    qe_row = qe_r[0]
    AP_scr[...] = (jnp.dot(W1qT_r[...], HqT, preferred_element_type=jnp.float32)
                   + (g3c * 1e-3) * qe_row + b1_r[...])

    @pl.when(j == 0)
    def _():
        ctx = ctx_r[0]
        h_ctx = jax.nn.relu(jnp.dot(ctx, cw1_r[...], preferred_element_type=jnp.float32) + cb1_r[...])
        h_ctx = jax.nn.relu(jnp.dot(h_ctx, cw2_r[...], preferred_element_type=jnp.float32) + cb2_r[...])
        pay = pay_r[0]
        h_pay = jax.nn.relu(jnp.dot(pay, pw1_r[...], preferred_element_type=jnp.float32) + pb1_r[...])
        h_pay = jax.nn.relu(jnp.dot(h_pay, pw2_r[...], preferred_element_type=jnp.float32) + pb2_r[...])
        HP_scr[...] = h_pay
        selev = selev_r[0]
        AC_scr[...] = (jnp.dot(h_ctx, W1c_r[...], preferred_element_type=jnp.float32)
                       - selev * (g3c.reshape(1, _HID) * 1e-3))

    cnt = jnp.int32(0)
    for n in range(_N):
        list_scr[cnt] = n
        cnt = cnt + list_s[0, 0, n]
    count = cnt

    pidx = jax.lax.broadcasted_iota(jnp.int32, (1, _P), 1) + j * _P
    prow = (pidx // _W).astype(jnp.float32)
    pcol = (pidx % _W).astype(jnp.float32)

    w2col = w2_r[...].reshape(_HID, 1)
    b2 = b2_s[0, 0]
    r2 = jnp.float32(_R * _R)

    S_scr[...] = jnp.full((_N, _P), _NEG, jnp.float32)
    HP2_scr[...] = jnp.zeros((_N, _HID), jnp.float32)

    AP = AP_scr[...]

    def _score(i):
        n = list_scr[i]
        rn = rows_s[0, 0, n].astype(jnp.float32)
        cn = cols_s[0, 0, n].astype(jnp.float32)
        drf = prow - rn
        dcf = pcol - cn
        d2 = drf * drf + dcf * dcf
        d = jnp.sqrt(d2)
        ok = d2 <= r2
        dkm = jnp.maximum(d, 1e-6)
        inv = 1.0 / dkm
        sinb = dcf * inv
        cosb = jnp.where(d > 0, -drf * inv, -1.0)
        acol = AC_scr[pl.ds(n, 1), :].reshape(_HID, 1)
        pre = (AP + acol
               + g0c * (dkm * (1.0 / _R)) + g1c * sinb + g2c * cosb)
        s = jnp.sum(jax.nn.relu(pre) * w2col, axis=0, keepdims=True) + b2
        return n, jnp.where(ok, s, _NEG)

    _U = 2

    def _stationU(it, carry):
        base = _U * it
        res = [_score(base + k) for k in range(_U)]
        n0, sm0 = res[0]
        S_scr[pl.ds(base, 1), :] = sm0
        HP2_scr[pl.ds(base, 1), :] = HP_scr[pl.ds(n0, 1), :]
        for k in range(1, _U):
            ik = base + k
            nk, smk = res[k]

            @pl.when(ik < count)
            def _(ik=ik, nk=nk, smk=smk):
                S_scr[pl.ds(ik, 1), :] = smk
                HP2_scr[pl.ds(ik, 1), :] = HP_scr[pl.ds(nk, 1), :]

        return carry

    jax.lax.fori_loop(0, (count + _U - 1) // _U, _stationU, jnp.int32(0))

    S = S_scr[...]
    m_raw = jnp.max(S, axis=0, keepdims=True)
    m = jnp.where(m_raw > 0.5 * _NEG, m_raw, 0.0)
    E = jnp.exp(S - m)
    den = jnp.sum(E, axis=0, keepdims=True)
    NumT = jax.lax.dot_general(HP2_scr[...], E, (((0,), (0,)), ((), ())),
                               preferred_element_type=jnp.float32)
    out_da[0] = NumT / jnp.maximum(den, 1e-30)
    out_cov[0] = (den > 0.5).astype(jnp.float32)


def kernel(F_grid, src_rows, src_cols, src_ctx, src_pay, src_valid,
           raw_elev_patch, src_elev,
           ctx_w1, ctx_b1, ctx_w2, ctx_b2, pay_w1, pay_b1, pay_w2, pay_b2,
           q_w, q_b, att_w1, att_b1, att_w2, att_b2):
    Ff = F_grid.reshape(_B, _C_LAT, _HW)
    qe = raw_elev_patch.reshape(_B, 1, _HW)
    rows2d = src_rows.astype(jnp.int32)
    valid2d = src_valid.astype(jnp.int32)
    lists = _route_call(rows2d, valid2d)
    lists3 = lists.reshape(_B, 1, _NBLK * _LROW)
    rows_i = rows2d.reshape(_B, 1, _N)
    cols_i = src_cols.astype(jnp.int32).reshape(_B, 1, _N)
    selev = src_elev.reshape(_B, _N, 1)
    qwT = q_w.T
    W1qT = att_w1[:_HID].T
    W1c = att_w1[_HID:2 * _HID]
    G4 = att_w1[2 * _HID:]
    w2row = att_w2.reshape(1, _HID)
    b1col = att_b1.reshape(_HID, 1)
    b2s = att_b2.reshape(1, 1)
    qbcol = q_b.reshape(_HID, 1)
    cb1r, cb2r = ctx_b1.reshape(1, _HID), ctx_b2.reshape(1, _HID)
    pb1r, pb2r = pay_b1.reshape(1, _HID), pay_b2.reshape(1, _HID)

    smem = pltpu.MemorySpace.SMEM
    grid = (_B, _NBLK)

    def full(shape):
        return pl.BlockSpec(shape, lambda b, j: (0,) * len(shape))

    in_specs = [
        pl.BlockSpec((1, 1, _N), lambda b, j: (b, 0, 0), memory_space=smem),
        pl.BlockSpec((1, 1, _N), lambda b, j: (b, 0, 0), memory_space=smem),
        pl.BlockSpec((1, 1, _LROW), lambda b, j: (b, 0, j), memory_space=smem),
        pl.BlockSpec((1, 1), lambda b, j: (0, 0), memory_space=smem),
        pl.BlockSpec((1, _C_LAT, _P), lambda b, j: (b, 0, j)),
        pl.BlockSpec((1, 1, _P), lambda b, j: (b, 0, j)),
        pl.BlockSpec((1, _N, 1), lambda b, j: (b, 0, 0)),
        pl.BlockSpec((1, _N, _C_CTX), lambda b, j: (b, 0, 0)),
        pl.BlockSpec((1, _N, _C_PAY), lambda b, j: (b, 0, 0)),
        full((_C_CTX, _HID)), full((1, _HID)),
        full((_HID, _HID)), full((1, _HID)),
        full((_C_PAY, _HID)), full((1, _HID)),
        full((_HID, _HID)), full((1, _HID)),
        full((_HID, _C_LAT)), full((_HID, 1)),
        full((_HID, _HID)), full((_HID, _HID)),
        full((4, _HID)), full((1, _HID)), full((_HID, 1)),
    ]
    out_specs = [
        pl.BlockSpec((1, _HID, _P), lambda b, j: (b, 0, j)),
        pl.BlockSpec((1, 1, _P), lambda b, j: (b, 0, j)),
    ]
    out_shape = [
        jax.ShapeDtypeStruct((_B, _HID, _HW), jnp.float32),
        jax.ShapeDtypeStruct((_B, 1, _HW), jnp.float32),
    ]

    da_flat, cov_flat = pl.pallas_call(
        _dafusion_block,
        grid=grid,
        in_specs=in_specs,
        out_specs=out_specs,
        out_shape=out_shape,
        scratch_shapes=[
            pltpu.VMEM((_N, _P), jnp.float32),
            pltpu.VMEM((_HID, _P), jnp.float32),
            pltpu.VMEM((_N, _HID), jnp.float32),
            pltpu.VMEM((_N, _HID), jnp.float32),
            pltpu.VMEM((_N, _HID), jnp.float32),
            pltpu.SMEM((_N,), jnp.int32),
        ],
        compiler_params=pltpu.CompilerParams(
            dimension_semantics=("parallel", "arbitrary")),
    )(rows_i, cols_i, lists3, b2s, Ff, qe, selev, src_ctx, src_pay,
      ctx_w1, cb1r, ctx_w2, cb2r, pay_w1, pb1r, pay_w2, pb2r,
      qwT, qbcol, W1qT, W1c, G4, w2row, b1col)

    da = da_flat.reshape(_B, _HID, _H, _W)
    cov = cov_flat.reshape(_B, 1, _H, _W)
    return da, cov


# speedup vs baseline: 1.2738x; 1.0031x over previous
import functools

import jax
import jax.numpy as jnp
from jax.experimental import pallas as pl
from jax.experimental.pallas import tpu as pltpu
from jax.experimental.pallas import tpu_sc as plsc

_B, _N, _H, _W = 4, 64, 128, 128
_HW = _H * _W
_C_LAT, _C_CTX, _C_PAY, _HID, _R = 256, 64, 32, 64, 16
_P = 1024
_ROWS = _P // _W
_NBLK = _HW // _P
_NEG = -1e30
_LROW = 128
_NITEM = _B * _NBLK


def _route_body(rows_hbm, valid_hbm, lists_hbm, rows_v, valid_v, list_v):
    wid = jax.lax.axis_index("s") * 2 + jax.lax.axis_index("c")
    for t in range(_NITEM // 32):
        item = wid * (_NITEM // 32) + t
        b = item // _NBLK
        j = item % _NBLK
        pltpu.sync_copy(rows_hbm.at[b], rows_v)
        pltpu.sync_copy(valid_hbm.at[b], valid_v)
        row_lo = _ROWS * j - _R
        row_hi = _ROWS * j + _ROWS - 1 + _R
        zeros = jnp.zeros((16,), jnp.int32)
        for z in range(_LROW // 16):
            list_v[pl.ds(16 * z, 16)] = zeros
        lo_v = jnp.full((16,), row_lo, jnp.int32)
        hi_v = jnp.full((16,), row_hi, jnp.int32)
        for cc in range(_N // 16):
            rv = rows_v[pl.ds(16 * cc, 16)]
            vv = valid_v[pl.ds(16 * cc, 16)]
            act = (rv >= lo_v) & (rv <= hi_v) & (vv > zeros)
            list_v[pl.ds(16 * cc, 16)] = jnp.where(act, 1, 0).astype(jnp.int32)
        pltpu.sync_copy(list_v, lists_hbm.at[b, j])


def _route_call(rows2d, valid2d):
    call = functools.partial(
        pl.kernel,
        mesh=plsc.VectorSubcoreMesh(core_axis_name="c", subcore_axis_name="s"),
        out_type=jax.ShapeDtypeStruct((_B, _NBLK, _LROW), jnp.int32),
        scratch_types=[
            pltpu.VMEM((_N,), jnp.int32),
            pltpu.VMEM((_N,), jnp.int32),
            pltpu.VMEM((_LROW,), jnp.int32),
        ],
    )(_route_body)
    return call(rows2d, valid2d)


def _dafusion_block(rows_s, cols_s, list_s, b2_s,
                    Ff_r, qe_r, selev_r, ctx_r, pay_r,
                    cw1_r, cb1_r, cw2_r, cb2_r,
                    pw1_r, pb1_r, pw2_r, pb2_r,
                    qwT_r, qb_r, W1qT_r, W1c_r, G4_r, w2_r, b1_r,
                    out_da, out_cov,
                    S_scr, AP_scr, AC_scr, HP_scr, HP2_scr, list_scr):
    j = pl.program_id(1)

    F_blk = Ff_r[0]
    HqT = jnp.dot(qwT_r[...], F_blk, preferred_element_type=jnp.float32)
    HqT = jax.nn.relu(HqT + qb_r[...])
    g0c = G4_r[0:1, :].reshape(_HID, 1)
    g1c = G4_r[1:2, :].reshape(_HID, 1)
    g2c = G4_r[2:3, :].reshape(_HID, 1)
    g3c = G4_r[3:4, :].reshape(_HID, 1)
    qe_row = qe_r[0]
    AP_scr[...] = (jnp.dot(W1qT_r[...], HqT, preferred_element_type=jnp.float32)
                   + (g3c * 1e-3) * qe_row + b1_r[...])

    @pl.when(j == 0)
    def _():
        ctx = ctx_r[0]
        h_ctx = jax.nn.relu(jnp.dot(ctx, cw1_r[...], preferred_element_type=jnp.float32) + cb1_r[...])
        h_ctx = jax.nn.relu(jnp.dot(h_ctx, cw2_r[...], preferred_element_type=jnp.float32) + cb2_r[...])
        pay = pay_r[0]
        h_pay = jax.nn.relu(jnp.dot(pay, pw1_r[...], preferred_element_type=jnp.float32) + pb1_r[...])
        h_pay = jax.nn.relu(jnp.dot(h_pay, pw2_r[...], preferred_element_type=jnp.float32) + pb2_r[...])
        HP_scr[...] = h_pay
        selev = selev_r[0]
        AC_scr[...] = (jnp.dot(h_ctx, W1c_r[...], preferred_element_type=jnp.float32)
                       - selev * (g3c.reshape(1, _HID) * 1e-3))

    cnt = jnp.int32(0)
    for n in range(_N):
        list_scr[cnt] = n
        cnt = cnt + list_s[0, 0, n]
    count = cnt

    pidx = jax.lax.broadcasted_iota(jnp.int32, (1, _P), 1) + j * _P
    prow = (pidx // _W).astype(jnp.float32)
    pcol = (pidx % _W).astype(jnp.float32)

    w2col = w2_r[...].reshape(_HID, 1)
    b2 = b2_s[0, 0]
    r2 = jnp.float32(_R * _R)

    S_scr[...] = jnp.full((_N, _P), _NEG, jnp.float32)
    HP2_scr[...] = jnp.zeros((_N, _HID), jnp.float32)

    AP = AP_scr[...]

    def _score(i):
        n = list_scr[i]
        rn = rows_s[0, 0, n].astype(jnp.float32)
        cn = cols_s[0, 0, n].astype(jnp.float32)
        drf = prow - rn
        dcf = pcol - cn
        d2 = drf * drf + dcf * dcf
        d = jnp.sqrt(d2)
        ok = d2 <= r2
        dkm = jnp.maximum(d, 1e-6)
        inv = 1.0 / dkm
        sinb = dcf * inv
        cosb = jnp.where(d > 0, -drf * inv, -1.0)
        acol = AC_scr[pl.ds(n, 1), :].reshape(_HID, 1)
        pre = (AP + acol
               + g0c * (dkm * (1.0 / _R)) + g1c * sinb + g2c * cosb)
        s = jnp.sum(jax.nn.relu(pre) * w2col, axis=0, keepdims=True) + b2
        return n, jnp.where(ok, s, _NEG)

    _U = 2

    def _stationU(it, carry):
        base = _U * it
        res = [_score(base + k) for k in range(_U)]
        n0, sm0 = res[0]
        S_scr[pl.ds(base, 1), :] = sm0
        HP2_scr[pl.ds(base, 1), :] = HP_scr[pl.ds(n0, 1), :]
        for k in range(1, _U):
            ik = base + k
            nk, smk = res[k]

            @pl.when(ik < count)
            def _(ik=ik, nk=nk, smk=smk):
                S_scr[pl.ds(ik, 1), :] = smk
                HP2_scr[pl.ds(ik, 1), :] = HP_scr[pl.ds(nk, 1), :]

        return carry

    jax.lax.fori_loop(0, (count + _U - 1) // _U, _stationU, jnp.int32(0))

    S = S_scr[...]
    m_raw = jnp.max(S, axis=0, keepdims=True)
    m = jnp.where(m_raw > 0.5 * _NEG, m_raw, 0.0)
    E = jnp.exp(S - m)
    den = jnp.sum(E, axis=0, keepdims=True)
    NumT = jax.lax.dot_general(HP2_scr[...], E, (((0,), (0,)), ((), ())),
                               preferred_element_type=jnp.float32)
    out_da[0] = NumT / jnp.maximum(den, 1e-30)
    out_cov[0] = (den > 0.5).astype(jnp.float32)


def kernel(F_grid, src_rows, src_cols, src_ctx, src_pay, src_valid,
           raw_elev_patch, src_elev,
           ctx_w1, ctx_b1, ctx_w2, ctx_b2, pay_w1, pay_b1, pay_w2, pay_b2,
           q_w, q_b, att_w1, att_b1, att_w2, att_b2):
    Ff = F_grid.reshape(_B, _C_LAT, _HW)
    qe = raw_elev_patch.reshape(_B, 1, _HW)
    rows2d = src_rows.astype(jnp.int32)
    valid2d = src_valid.astype(jnp.int32)
    lists = _route_call(rows2d, valid2d)
    lists3 = lists.reshape(_B, 1, _NBLK * _LROW)
    rows_i = rows2d.reshape(_B, 1, _N)
    cols_i = src_cols.astype(jnp.int32).reshape(_B, 1, _N)
    selev = src_elev.reshape(_B, _N, 1)
    qwT = q_w.T
    W1qT = att_w1[:_HID].T
    W1c = att_w1[_HID:2 * _HID]
    G4 = att_w1[2 * _HID:]
    w2row = att_w2.reshape(1, _HID)
    b1col = att_b1.reshape(_HID, 1)
    b2s = att_b2.reshape(1, 1)
    qbcol = q_b.reshape(_HID, 1)
    cb1r, cb2r = ctx_b1.reshape(1, _HID), ctx_b2.reshape(1, _HID)
    pb1r, pb2r = pay_b1.reshape(1, _HID), pay_b2.reshape(1, _HID)

    smem = pltpu.MemorySpace.SMEM
    grid = (_B, _NBLK)

    def full(shape):
        return pl.BlockSpec(shape, lambda b, j: (0,) * len(shape))

    in_specs = [
        pl.BlockSpec((1, 1, _N), lambda b, j: (b, 0, 0), memory_space=smem),
        pl.BlockSpec((1, 1, _N), lambda b, j: (b, 0, 0), memory_space=smem),
        pl.BlockSpec((1, 1, _LROW), lambda b, j: (b, 0, j), memory_space=smem),
        pl.BlockSpec((1, 1), lambda b, j: (0, 0), memory_space=smem),
        pl.BlockSpec((1, _C_LAT, _P), lambda b, j: (b, 0, j)),
        pl.BlockSpec((1, 1, _P), lambda b, j: (b, 0, j)),
        pl.BlockSpec((1, _N, 1), lambda b, j: (b, 0, 0)),
        pl.BlockSpec((1, _N, _C_CTX), lambda b, j: (b, 0, 0)),
        pl.BlockSpec((1, _N, _C_PAY), lambda b, j: (b, 0, 0)),
        full((_C_CTX, _HID)), full((1, _HID)),
        full((_HID, _HID)), full((1, _HID)),
        full((_C_PAY, _HID)), full((1, _HID)),
        full((_HID, _HID)), full((1, _HID)),
        full((_HID, _C_LAT)), full((_HID, 1)),
        full((_HID, _HID)), full((_HID, _HID)),
        full((4, _HID)), full((1, _HID)), full((_HID, 1)),
    ]
    out_specs = [
        pl.BlockSpec((1, _HID, _P), lambda b, j: (b, 0, j)),
        pl.BlockSpec((1, 1, _P), lambda b, j: (b, 0, j)),
    ]
    out_shape = [
        jax.ShapeDtypeStruct((_B, _HID, _HW), jnp.float32),
        jax.ShapeDtypeStruct((_B, 1, _HW), jnp.float32),
    ]

    da_flat, cov_flat = pl.pallas_call(
        _dafusion_block,
        grid=grid,
        in_specs=in_specs,
        out_specs=out_specs,
        out_shape=out_shape,
        scratch_shapes=[
            pltpu.VMEM((_N, _P), jnp.float32),
            pltpu.VMEM((_HID, _P), jnp.float32),
            pltpu.VMEM((_N, _HID), jnp.float32),
            pltpu.VMEM((_N, _HID), jnp.float32),
            pltpu.VMEM((_N, _HID), jnp.float32),
            pltpu.SMEM((_N,), jnp.int32),
        ],
        compiler_params=pltpu.CompilerParams(
            dimension_semantics=("parallel", "arbitrary")),
    )(rows_i, cols_i, lists3, b2s, Ff, qe, selev, src_ctx, src_pay,
      ctx_w1, cb1r, ctx_w2, cb2r, pay_w1, pb1r, pay_w2, pb2r,
      qwT, qbcol, W1qT, W1c, G4, w2row, b1col)

    da = da_flat.reshape(_B, _HID, _H, _W)
    cov = cov_flat.reshape(_B, 1, _H, _W)
    return da, cov
